# Initial kernel scaffold; baseline (speedup 1.0000x reference)
import jax
import jax.numpy as jnp
from jax.experimental import pallas as pl


def kernel(x, index, supports_01):
    raise NotImplementedError("write your pallas kernel here")


# speedup vs baseline: 1.1053x; 1.1053x over previous
import functools

import jax
import jax.numpy as jnp
from jax import lax
from jax.experimental import pallas as pl
from jax.experimental.pallas import tpu as pltpu
from jax.experimental.pallas import tpu_sc as plsc

BATCH = 16384
HIST = 20
EMBED_DIM = 32
N = BATCH * HIST

NUM_CORES = 2
NUM_SUBCORES = 16
NUM_WORKERS = NUM_CORES * NUM_SUBCORES
ROWS_PER_WORKER = N // NUM_WORKERS
CHUNK = 1024
NUM_CHUNKS = ROWS_PER_WORKER // CHUNK
LANES = 16


def _sc_fused(table_hbm, idx_hbm, x_hbm, out_hbm, idx_v, rows_v, x_v,
              sem_g, sem_x):
    wid = lax.axis_index("s") * NUM_CORES + lax.axis_index("c")
    base = wid * ROWS_PER_WORKER

    @pl.loop(0, NUM_CHUNKS)
    def _(ci):
        start = base + ci * CHUNK
        pltpu.sync_copy(idx_hbm.at[pl.ds(start, CHUNK)], idx_v)
        cg = pltpu.async_copy(table_hbm.at[idx_v], rows_v, sem_g)
        cx = pltpu.async_copy(x_hbm.at[pl.ds(start, CHUNK)], x_v, sem_x)
        cx.wait()
        cg.wait()

        @pl.loop(0, CHUNK)
        def _(r):
            for c in range(0, EMBED_DIM, LANES):
                xv = x_v[r, pl.ds(c, LANES)]
                g = rows_v[r, pl.ds(c, LANES)]
                rows_v[r, pl.ds(c, LANES)] = g / (1.0 + jnp.exp(-2.0 * xv))

        pltpu.sync_copy(rows_v, out_hbm.at[pl.ds(start, CHUNK)])


def kernel(x, index, supports_01):
    xf = x.reshape(N, EMBED_DIM)
    idx = index.reshape(N).astype(jnp.int32)

    fused = functools.partial(
        pl.kernel,
        out_type=jax.ShapeDtypeStruct((N, EMBED_DIM), jnp.float32),
        mesh=plsc.VectorSubcoreMesh(core_axis_name="c", subcore_axis_name="s"),
        scratch_types=[
            pltpu.VMEM((CHUNK,), jnp.int32),
            pltpu.VMEM((CHUNK, EMBED_DIM), jnp.float32),
            pltpu.VMEM((CHUNK, EMBED_DIM), jnp.float32),
            pltpu.SemaphoreType.DMA,
            pltpu.SemaphoreType.DMA,
        ],
        compiler_params=pltpu.CompilerParams(use_tc_tiling_on_sc=False),
    )(_sc_fused)

    out = fused(supports_01, idx, xf)
    return out.reshape(BATCH, HIST, EMBED_DIM)
